# Initial kernel scaffold; baseline (speedup 1.0000x reference)
import jax
import jax.numpy as jnp
from jax.experimental import pallas as pl


def kernel(label_ids, logits):
    raise NotImplementedError("write your pallas kernel here")


# speedup vs baseline: 55.7096x; 55.7096x over previous
import functools

import jax
import jax.numpy as jnp
from jax import lax
from jax.experimental import pallas as pl
from jax.experimental.pallas import tpu as pltpu

_K = 5
_NEG = float("-inf")


def _loss_body(lab_ref, x_ref, out_ref, acc_ref):
    i = pl.program_id(0)

    @pl.when(i == 0)
    def _init():
        acc_ref[0] = jnp.float32(0.0)
        acc_ref[1] = jnp.float32(0.0)

    x = x_ref[...]
    lab = lab_ref[...]
    r, v_dim = x.shape
    col = lax.broadcasted_iota(jnp.int32, (r, v_dim), 1)

    m = jnp.max(x, axis=1, keepdims=True)
    z = jnp.sum(jnp.exp(x - m), axis=1, keepdims=True)
    l_lab = jnp.max(jnp.where(col == lab, x, _NEG), axis=1, keepdims=True)

    vs = []
    cur = x
    top = m
    for t in range(_K):
        if t:
            top = jnp.max(cur, axis=1, keepdims=True)
        vs.append(top)
        if t != _K - 1:
            cur = jnp.where(cur == top, _NEG, cur)

    pp = jnp.exp(l_lab - m) / z
    e0 = jnp.exp(pp)
    s = e0
    kc = jnp.zeros((r, 1), jnp.float32)
    for vt in vs:
        keep = vt != l_lab
        tv = jnp.exp(vt - m) / z
        s = s + jnp.where(keep, jnp.exp(tv), 0.0)
        kc = kc + keep.astype(jnp.float32)
    mini0 = e0 / s
    per_pos = (1.0 - (kc + 1.0) * mini0) / kc
    validf = ((lab != 0) & (vs[0] != l_lab)).astype(jnp.float32)
    acc_ref[0] += jnp.sum(per_pos * validf)
    acc_ref[1] += jnp.sum(validf)

    @pl.when(i == pl.num_programs(0) - 1)
    def _fin():
        cnt = acc_ref[1]
        out_ref[0, 0] = jnp.where(cnt > 0.0,
                                  acc_ref[0] / jnp.maximum(cnt, 1.0),
                                  jnp.float32(0.0))


def kernel(label_ids, logits):
    b, s, v = logits.shape
    n = b * s
    x = logits.reshape(n, v)
    lab = label_ids.reshape(n, 1)
    r = 8
    out = pl.pallas_call(
        _loss_body,
        grid=(n // r,),
        in_specs=[
            pl.BlockSpec((r, 1), lambda i: (i, 0)),
            pl.BlockSpec((r, v), lambda i: (i, 0)),
        ],
        out_specs=pl.BlockSpec(memory_space=pltpu.SMEM),
        out_shape=jax.ShapeDtypeStruct((1, 1), jnp.float32),
        scratch_shapes=[pltpu.SMEM((2,), jnp.float32)],
    )(lab, x)
    return out[0, 0]
